# Initial kernel scaffold; baseline (speedup 1.0000x reference)
import jax
import jax.numpy as jnp
from jax.experimental import pallas as pl


def kernel(x, alpha1_raw):
    raise NotImplementedError("write your pallas kernel here")


# speedup vs baseline: 11.3198x; 11.3198x over previous
import functools

import jax
import jax.numpy as jnp
from jax.experimental import pallas as pl
from jax.experimental.pallas import tpu as pltpu


def _time_mix_weights(T, dtype):
    tg = jnp.linspace(-1.0, 1.0, T, dtype=dtype)
    iy = (tg + 1.0) * 0.5 * (T - 1)
    iy0 = jnp.floor(iy)
    wy1 = iy - iy0
    wy0 = 1.0 - wy1
    v0 = (iy0 >= 0) & (iy0 <= T - 1)
    v1 = (iy0 + 1.0 >= 0) & (iy0 + 1.0 <= T - 1)
    wy0 = jnp.where(v0, wy0, jnp.zeros_like(wy0))
    wy1 = jnp.where(v1, wy1, jnp.zeros_like(wy1))
    s = iy0.astype(jnp.int32) == jnp.arange(T, dtype=jnp.int32)
    zero = jnp.zeros_like(wy0)
    wA = jnp.where(s, zero, wy0)
    wB = jnp.where(s, wy0, wy1)
    wC = jnp.where(s, wy1, zero)
    return jnp.stack([wA, wB, wC], axis=1)


def _freq_warp_matrix(alpha, D, dtype):
    f = jnp.linspace(0.0, 1.0, D, dtype=dtype)
    warped = f ** alpha
    gx = warped * 2.0 - 1.0
    ix = (gx + 1.0) * 0.5 * (D - 1)
    ix0 = jnp.floor(ix)
    wx1 = ix - ix0
    wx0 = 1.0 - wx1
    v0 = (ix0 >= 0) & (ix0 <= D - 1)
    v1 = (ix0 + 1.0 >= 0) & (ix0 + 1.0 <= D - 1)
    wx0 = jnp.where(v0, wx0, jnp.zeros_like(wx0))
    wx1 = jnp.where(v1, wx1, jnp.zeros_like(wx1))
    i0 = jnp.clip(ix0, 0, D - 1).astype(jnp.int32)
    i1 = jnp.clip(ix0 + 1.0, 0, D - 1).astype(jnp.int32)
    k = jnp.arange(D, dtype=jnp.int32)[:, None]
    M = (wx0[None, :] * (k == i0[None, :]).astype(dtype)
         + wx1[None, :] * (k == i1[None, :]).astype(dtype))
    return M


def _tc_body(x_ref, w_ref, m_ref, o_ref, prev_ref, plast_ref):
    i = pl.program_id(0)

    @pl.when(i > 0)
    def _compute():
        prev = prev_ref[...]
        xm = jnp.concatenate([plast_ref[...], prev[:-1, :]], axis=0)
        xp = jnp.concatenate([prev[1:, :], x_ref[0:1, :]], axis=0)
        wA = w_ref[:, 0:1]
        wB = w_ref[:, 1:2]
        wC = w_ref[:, 2:3]
        mixed = wA * xm + wB * prev + wC * xp
        o_ref[...] = jax.lax.dot_general(
            mixed, m_ref[...], (((1,), (0,)), ((), ())),
            precision=jax.lax.Precision.HIGHEST,
            preferred_element_type=jnp.float32)
        plast_ref[...] = prev[-1:, :]

    @pl.when(i == 0)
    def _init():
        plast_ref[...] = jnp.zeros_like(plast_ref)

    prev_ref[...] = x_ref[...]


def _tc_warp(x2, w, M, block_t):
    T, D = x2.shape
    nb = T // block_t
    return pl.pallas_call(
        _tc_body,
        grid=(nb + 1,),
        in_specs=[
            pl.BlockSpec((block_t, D), lambda i: (jnp.minimum(i, nb - 1), 0)),
            pl.BlockSpec((block_t, 3), lambda i: (jnp.maximum(i - 1, 0), 0)),
            pl.BlockSpec((D, D), lambda i: (0, 0)),
        ],
        out_specs=pl.BlockSpec((block_t, D), lambda i: (jnp.maximum(i - 1, 0), 0)),
        out_shape=jax.ShapeDtypeStruct((T, D), x2.dtype),
        scratch_shapes=[
            pltpu.VMEM((block_t, D), x2.dtype),
            pltpu.VMEM((1, D), x2.dtype),
        ],
    )(x2, w, M)


def kernel(x, alpha1_raw):
    B, T, D = x.shape
    assert B == 1
    x2 = x.reshape(T, D)
    alpha = jnp.reshape(alpha1_raw, ())
    w = _time_mix_weights(T, x.dtype)
    M = _freq_warp_matrix(alpha, D, x.dtype)
    out = _tc_warp(x2, w, M, block_t=1200)
    return out.reshape(B, T, D)
